# Initial kernel scaffold; baseline (speedup 1.0000x reference)
import jax
import jax.numpy as jnp
from jax.experimental import pallas as pl


def kernel(input_ids, table):
    raise NotImplementedError("write your pallas kernel here")


# speedup vs baseline: 1.0989x; 1.0989x over previous
import functools

import jax
import jax.numpy as jnp
from jax import lax
from jax.experimental import pallas as pl
from jax.experimental.pallas import tpu as pltpu
from jax.experimental.pallas import tpu_sc as plsc

VOCAB = 1000000
DIM = 32
B = 16384
L = 50
N = B * L

NC = 2
NS = 16
NW = NC * NS
PER_W = N // NW
SUB = 128
KSUB = 10
CHUNK = SUB * KSUB
NCH = PER_W // CHUNK


def _embed_body(ids_hbm, table_hbm, out_hbm, idx_v, rows_v, sem):
    wid = lax.axis_index("s") * NC + lax.axis_index("c")

    def chunk_body(g, carry):
        chunk = wid * NCH + g
        pltpu.sync_copy(ids_hbm.at[chunk], idx_v)
        cps = [
            pltpu.async_copy(
                table_hbm.at[idx_v.at[j]],
                rows_v.at[pl.ds(j * SUB, SUB)],
                sem,
            )
            for j in range(KSUB)
        ]
        for cp in cps:
            cp.wait()
        pltpu.sync_copy(rows_v, out_hbm.at[pl.ds(chunk * CHUNK, CHUNK)])
        return carry

    lax.fori_loop(0, NCH, chunk_body, 0)


@jax.jit
def kernel(input_ids, table):
    ids = input_ids.reshape(NW * NCH, KSUB, SUB).astype(jnp.int32)
    mesh = plsc.VectorSubcoreMesh(core_axis_name="c", subcore_axis_name="s")
    out = pl.kernel(
        _embed_body,
        mesh=mesh,
        out_type=jax.ShapeDtypeStruct((N, DIM), jnp.float32),
        scratch_types=[
            pltpu.VMEM((KSUB, SUB), jnp.int32),
            pltpu.VMEM((CHUNK, DIM), jnp.float32),
            pltpu.SemaphoreType.DMA,
        ],
        compiler_params=pltpu.CompilerParams(use_tc_tiling_on_sc=False),
    )(ids, table)
    return out.reshape(B, L, DIM)


# speedup vs baseline: 1.1081x; 1.0084x over previous
import jax
import jax.numpy as jnp
from jax import lax
from jax.experimental import pallas as pl
from jax.experimental.pallas import tpu as pltpu
from jax.experimental.pallas import tpu_sc as plsc

VOCAB = 1000000
DIM = 32
B = 16384
L = 50
N = B * L

NC = 2
NS = 16
NW = NC * NS
PER_W = N // NW
SUB = 128
KSUB = 10
CHUNK = SUB * KSUB
NCH = PER_W // CHUNK


def _embed_body(ids_hbm, table_hbm, out_hbm, idx_v, rows_v, gsem, wsem0, wsem1):
    wid = lax.axis_index("s") * NC + lax.axis_index("c")
    base = wid * NCH
    wsems = (wsem0, wsem1)

    def do_chunk(g, p):
        chunk = base + g
        pltpu.sync_copy(ids_hbm.at[chunk], idx_v.at[p])

        @pl.when(g >= 2)
        def _():
            pltpu.make_async_copy(
                rows_v.at[p],
                out_hbm.at[pl.ds((chunk - 2) * CHUNK, CHUNK)],
                wsems[p],
            ).wait()

        cps = [
            pltpu.async_copy(
                table_hbm.at[idx_v.at[p, j]],
                rows_v.at[p, pl.ds(j * SUB, SUB)],
                gsem,
            )
            for j in range(KSUB)
        ]
        for cp in cps:
            cp.wait()

        pltpu.async_copy(
            rows_v.at[p],
            out_hbm.at[pl.ds(chunk * CHUNK, CHUNK)],
            wsems[p],
        )

    def pair_body(h, carry):
        do_chunk(2 * h, 0)
        do_chunk(2 * h + 1, 1)
        return carry

    lax.fori_loop(0, NCH // 2, pair_body, 0)

    pltpu.make_async_copy(
        rows_v.at[0],
        out_hbm.at[pl.ds((base + NCH - 2) * CHUNK, CHUNK)],
        wsem0,
    ).wait()
    pltpu.make_async_copy(
        rows_v.at[1],
        out_hbm.at[pl.ds((base + NCH - 1) * CHUNK, CHUNK)],
        wsem1,
    ).wait()


@jax.jit
def kernel(input_ids, table):
    ids = input_ids.reshape(NW * NCH, KSUB, SUB).astype(jnp.int32)
    mesh = plsc.VectorSubcoreMesh(core_axis_name="c", subcore_axis_name="s")
    out = pl.kernel(
        _embed_body,
        mesh=mesh,
        out_type=jax.ShapeDtypeStruct((N, DIM), jnp.float32),
        scratch_types=[
            pltpu.VMEM((2, KSUB, SUB), jnp.int32),
            pltpu.VMEM((2, CHUNK, DIM), jnp.float32),
            pltpu.SemaphoreType.DMA,
            pltpu.SemaphoreType.DMA,
            pltpu.SemaphoreType.DMA,
        ],
        compiler_params=pltpu.CompilerParams(use_tc_tiling_on_sc=False),
    )(ids, table)
    return out.reshape(B, L, DIM)


# speedup vs baseline: 1.1131x; 1.0045x over previous
import jax
import jax.numpy as jnp
from jax import lax
from jax.experimental import pallas as pl
from jax.experimental.pallas import tpu as pltpu
from jax.experimental.pallas import tpu_sc as plsc

VOCAB = 1000000
DIM = 32
B = 16384
L = 50
N = B * L

NC = 2
NS = 16
NW = NC * NS
PER_W = N // NW
CHUNK = 1280
NCH = PER_W // CHUNK


def _embed_body(ids_hbm, table_hbm, out_hbm, idx_v, rows_v,
                gsem0, gsem1, wsem0, wsem1):
    wid = lax.axis_index("s") * NC + lax.axis_index("c")
    base = wid * NCH
    gsems = (gsem0, gsem1)
    wsems = (wsem0, wsem1)

    pltpu.sync_copy(ids_hbm.at[base], idx_v.at[0])
    pltpu.async_copy(table_hbm.at[idx_v.at[0]], rows_v.at[0], gsem0)

    for g in range(NCH):
        p = g % 2
        q = 1 - p
        if g + 1 < NCH:
            pltpu.sync_copy(ids_hbm.at[base + g + 1], idx_v.at[q])
            if g >= 1:
                pltpu.make_async_copy(
                    rows_v.at[q],
                    out_hbm.at[pl.ds((base + g - 1) * CHUNK, CHUNK)],
                    wsems[q],
                ).wait()
            pltpu.async_copy(table_hbm.at[idx_v.at[q]], rows_v.at[q], gsems[q])
        pltpu.make_async_copy(
            table_hbm.at[idx_v.at[p]], rows_v.at[p], gsems[p]
        ).wait()
        pltpu.async_copy(
            rows_v.at[p],
            out_hbm.at[pl.ds((base + g) * CHUNK, CHUNK)],
            wsems[p],
        )

    pltpu.make_async_copy(
        rows_v.at[0],
        out_hbm.at[pl.ds((base + NCH - 2) * CHUNK, CHUNK)],
        wsem0,
    ).wait()
    pltpu.make_async_copy(
        rows_v.at[1],
        out_hbm.at[pl.ds((base + NCH - 1) * CHUNK, CHUNK)],
        wsem1,
    ).wait()


@jax.jit
def kernel(input_ids, table):
    ids = input_ids.reshape(NW * NCH, CHUNK).astype(jnp.int32)
    mesh = plsc.VectorSubcoreMesh(core_axis_name="c", subcore_axis_name="s")
    out = pl.kernel(
        _embed_body,
        mesh=mesh,
        out_type=jax.ShapeDtypeStruct((N, DIM), jnp.float32),
        scratch_types=[
            pltpu.VMEM((2, CHUNK), jnp.int32),
            pltpu.VMEM((2, CHUNK, DIM), jnp.float32),
            pltpu.SemaphoreType.DMA,
            pltpu.SemaphoreType.DMA,
            pltpu.SemaphoreType.DMA,
            pltpu.SemaphoreType.DMA,
        ],
        compiler_params=pltpu.CompilerParams(use_tc_tiling_on_sc=False),
    )(ids, table)
    return out.reshape(B, L, DIM)


# speedup vs baseline: 1.4117x; 1.2683x over previous
import jax
import jax.numpy as jnp
from jax import lax
from jax.experimental import pallas as pl
from jax.experimental.pallas import tpu as pltpu
from jax.experimental.pallas import tpu_sc as plsc

VOCAB = 1000000
DIM = 32
B = 16384
L = 50
N = B * L

NC = 2
NS = 16
NW = NC * NS
BPW = B // NW
BBLK = 128
NBB = BPW // BBLK
NBLK = L * NBB


def _embed_body(ids_hbm, table_hbm, out_hbm, idx_v, rows_v, tr_v,
                gsem0, gsem1, wsem0, wsem1):
    wid = lax.axis_index("s") * NC + lax.axis_index("c")
    b0w = wid * BPW
    gsems = (gsem0, gsem1)
    wsems = (wsem0, wsem1)
    iota = lax.iota(jnp.int32, 16)

    pltpu.sync_copy(ids_hbm.at[:, pl.ds(b0w, BPW)], idx_v)

    def idx_slice(it):
        l = it // NBB
        bb = it % NBB
        return idx_v.at[l, pl.ds(bb * BBLK, BBLK)]

    def out_slice(it):
        l = it // NBB
        bb = it % NBB
        return out_hbm.at[l, :, pl.ds(b0w + bb * BBLK, BBLK)]

    pltpu.async_copy(table_hbm.at[idx_slice(0)], rows_v.at[0], gsem0)

    def steady(it, p):
        q = 1 - p
        @pl.when(it + 1 < NBLK)
        def _():
            pltpu.async_copy(table_hbm.at[idx_slice(it + 1)], rows_v.at[q],
                             gsems[q])
        pltpu.make_async_copy(
            table_hbm.at[idx_slice(it)], rows_v.at[p], gsems[p]
        ).wait()
        @pl.when(it >= 2)
        def _():
            pltpu.make_async_copy(tr_v.at[p], out_slice(it - 2),
                                  wsems[p]).wait()
        for d in range(DIM):
            dcol = jnp.full((16,), d, jnp.int32)
            for s in range(BBLK // 16):
                seg = plsc.load_gather(rows_v.at[p], [s * 16 + iota, dcol])
                tr_v[p, d, pl.ds(s * 16, 16)] = seg
        pltpu.async_copy(tr_v.at[p], out_slice(it), wsems[p])

    def pair_body(h, carry):
        steady(2 * h, 0)
        steady(2 * h + 1, 1)
        return carry

    lax.fori_loop(0, NBLK // 2, pair_body, 0)

    pltpu.make_async_copy(tr_v.at[0], out_slice(NBLK - 2), wsem0).wait()
    pltpu.make_async_copy(tr_v.at[1], out_slice(NBLK - 1), wsem1).wait()


@jax.jit
def kernel(input_ids, table):
    ids_t = input_ids.T.astype(jnp.int32)
    mesh = plsc.VectorSubcoreMesh(core_axis_name="c", subcore_axis_name="s")
    out = pl.kernel(
        _embed_body,
        mesh=mesh,
        out_type=jax.ShapeDtypeStruct((L, DIM, B), jnp.float32),
        scratch_types=[
            pltpu.VMEM((L, BPW), jnp.int32),
            pltpu.VMEM((2, BBLK, DIM), jnp.float32),
            pltpu.VMEM((2, DIM, BBLK), jnp.float32),
            pltpu.SemaphoreType.DMA,
            pltpu.SemaphoreType.DMA,
            pltpu.SemaphoreType.DMA,
            pltpu.SemaphoreType.DMA,
        ],
        compiler_params=pltpu.CompilerParams(
            use_tc_tiling_on_sc=False, needs_layout_passes=False),
    )(ids_t, table)
    return jnp.transpose(out, (2, 0, 1))
